# baseline (speedup 1.0000x reference)
import functools

import jax
import jax.numpy as jnp
from jax import lax
from jax.experimental import pallas as pl
from jax.experimental.pallas import tpu as pltpu


def _round_up(x, m):
    return (x + m - 1) // m * m


def _body(x1_ref, x2_ref, wlwt_ref, blw_ref, wcat_ref, wsum_ref, b0_ref,
          w1_ref, b1_ref, out_ref, wbig_ref, wlw_s, colv_s, *, d, groups, tb):
    f32 = jnp.float32
    bf16 = jnp.bfloat16
    md = groups * d

    @pl.when(pl.program_id(0) == 0)
    def _build_weights():
        t_ab = jnp.transpose(wcat_ref[:, :md].astype(bf16))
        t_s = jnp.transpose(wcat_ref[:, md:2 * md].astype(bf16))
        t_sum = jnp.transpose(wsum_ref[:, :groups].astype(bf16))
        wbig_ref[:md, :d] = t_ab
        wbig_ref[:md, d:] = t_s
        wbig_ref[md:2 * md, :d] = t_s
        wbig_ref[md:2 * md, d:] = jnp.zeros((md, d), bf16)
        wbig_ref[2 * md:, :d] = t_sum
        wbig_ref[2 * md:, d:] = t_sum
        wlw_s[...] = jnp.transpose(wlwt_ref[...])
        colv_s[:d] = jnp.transpose(blw_ref[...])
        colv_s[d:] = jnp.transpose(b0_ref[...])

    blw = colv_s[:d]
    wlwt = wlw_s[...]

    def frontend(x_ref):
        y = lax.dot_general(wlwt, x_ref[...], (((1,), (1,)), ((), ())),
                            preferred_element_type=f32) + blw
        ss = jnp.sum(y * y, axis=0, keepdims=True)
        return y * lax.rsqrt(jnp.maximum(ss, 1e-24))

    at = frontend(x1_ref)
    bt = frontend(x2_ref)
    ct = jnp.concatenate([at, bt], axis=0).astype(bf16)

    big = jnp.dot(wbig_ref[...], ct, preferred_element_type=f32)

    sumterm = big[2 * md:2 * md + groups]

    nchunk = 4
    gc = groups // nchunk
    h_parts = []
    for c in range(nchunk):
        p1 = big[c * gc * d:(c + 1) * gc * d].reshape(gc, d, tb)
        p2 = big[md + c * gc * d:md + (c + 1) * gc * d].reshape(gc, d, tb)
        h_parts.append(
            jnp.sum(p1 * bt[None, :, :] + p2 * at[None, :, :], axis=1))
    b0c = colv_s[d:d + groups]
    h = jnp.concatenate(h_parts, axis=0) + sumterm + b0c
    h = jnp.maximum(h, 0.0)

    w1c = w1_ref[:groups, :1]
    score = jnp.sum(h * w1c, axis=0, keepdims=True) + b1_ref[0:1, 0:1]
    out_ref[...] = jnp.transpose(score)


def kernel(x1, x2, wlw_t, blw_r, wcat, wsum_t, b0p, pool, w1p, b1p):
    batch, in_dim = x1.shape
    d = wlw_t.shape[1]
    md_pad = wcat.shape[1] // 3
    groups = md_pad // d
    f32 = jnp.float32

    tb = 512
    b_pad = _round_up(batch, tb)
    pad = b_pad - batch
    x1 = x1.astype(f32)
    x2 = x2.astype(f32)
    if pad:
        x1 = jnp.pad(x1, ((0, pad), (0, 0)))
        x2 = jnp.pad(x2, ((0, pad), (0, 0)))

    body = functools.partial(_body, d=d, groups=groups, tb=tb)
    row_spec = pl.BlockSpec((tb, in_dim), lambda i: (i, 0))
    full = lambda w: pl.BlockSpec(tuple(w.shape), lambda i: (0, 0))

    out = pl.pallas_call(
        body,
        out_shape=jax.ShapeDtypeStruct((b_pad, 1), f32),
        grid=(b_pad // tb,),
        in_specs=[row_spec, row_spec, full(wlw_t), full(blw_r), full(wcat),
                  full(wsum_t), full(b0p), full(w1p), full(b1p)],
        out_specs=pl.BlockSpec((tb, 1), lambda i: (i, 0)),
        scratch_shapes=[pltpu.VMEM((2 * md_pad + groups, 2 * d), jnp.bfloat16),
                        pltpu.VMEM((d, in_dim), f32),
                        pltpu.VMEM((d + b0p.shape[1], 1), f32)],
        compiler_params=pltpu.CompilerParams(
            dimension_semantics=("arbitrary",),
            vmem_limit_bytes=100 << 20,
        ),
    )(x1, x2, wlw_t, blw_r, wcat, wsum_t, b0p, w1p, b1p)
    return out[:batch]


# speedup vs baseline: 1.0788x; 1.0788x over previous
import functools

import jax
import jax.numpy as jnp
from jax import lax
from jax.experimental import pallas as pl
from jax.experimental.pallas import tpu as pltpu


def _round_up(x, m):
    return (x + m - 1) // m * m


def _body(x1_ref, x2_ref, wlwt_ref, blw_ref, wcat_ref, wsum_ref, b0_ref,
          w1_ref, b1_ref, out_ref, wbig_ref, wlw_s, colv_s, *, d, groups, tb):
    f32 = jnp.float32
    bf16 = jnp.bfloat16
    md = groups * d

    @pl.when(pl.program_id(0) == 0)
    def _build_weights():
        t_ab = jnp.transpose(wcat_ref[:, :md].astype(bf16))
        t_s = jnp.transpose(wcat_ref[:, md:2 * md].astype(bf16))
        t_sum = jnp.transpose(wsum_ref[:, :groups].astype(bf16))
        wbig_ref[:md, :d] = t_ab
        wbig_ref[:md, d:] = t_s
        wbig_ref[md:2 * md, :d] = t_s
        wbig_ref[md:2 * md, d:] = jnp.zeros((md, d), bf16)
        wbig_ref[2 * md:, :d] = t_sum
        wbig_ref[2 * md:, d:] = t_sum
        wlw_s[...] = jnp.transpose(wlwt_ref[...])
        colv_s[:d] = jnp.transpose(blw_ref[...])
        colv_s[d:] = jnp.transpose(b0_ref[...])

    blw = colv_s[:d]
    wlwt = wlw_s[...]

    def frontend(x_ref):
        y = lax.dot_general(wlwt, x_ref[...], (((1,), (1,)), ((), ())),
                            preferred_element_type=f32) + blw
        ss = jnp.sum(y * y, axis=0, keepdims=True)
        return y * lax.rsqrt(jnp.maximum(ss, 1e-24))

    at = frontend(x1_ref)
    bt = frontend(x2_ref)
    ct = jnp.concatenate([at, bt], axis=0).astype(bf16)

    big = jnp.dot(wbig_ref[...], ct, preferred_element_type=f32)

    p1 = big[:md].reshape(groups, d, tb)
    p2 = big[md:2 * md].reshape(groups, d, tb)
    sumterm = big[2 * md:2 * md + groups]

    s3 = p1 * bt[None, :, :] + p2 * at[None, :, :]
    b0c = colv_s[d:d + groups]
    h = jnp.sum(s3, axis=1) + sumterm + b0c
    h = jnp.maximum(h, 0.0)

    w1c = w1_ref[:groups, :1]
    score = jnp.sum(h * w1c, axis=0, keepdims=True) + b1_ref[0:1, 0:1]
    out_ref[...] = jnp.broadcast_to(score, (8, tb))


def kernel(x1, x2, wlw_t, blw_r, wcat, wsum_t, b0p, pool, w1p, b1p):
    batch, in_dim = x1.shape
    d = wlw_t.shape[1]
    md_pad = wcat.shape[1] // 3
    groups = md_pad // d
    f32 = jnp.float32

    tb = 256
    b_pad = _round_up(batch, tb)
    pad = b_pad - batch
    x1 = x1.astype(f32)
    x2 = x2.astype(f32)
    if pad:
        x1 = jnp.pad(x1, ((0, pad), (0, 0)))
        x2 = jnp.pad(x2, ((0, pad), (0, 0)))

    body = functools.partial(_body, d=d, groups=groups, tb=tb)
    row_spec = pl.BlockSpec((tb, in_dim), lambda i: (i, 0))
    full = lambda w: pl.BlockSpec(tuple(w.shape), lambda i: (0, 0))

    out = pl.pallas_call(
        body,
        out_shape=jax.ShapeDtypeStruct((8, b_pad), f32),
        grid=(b_pad // tb,),
        in_specs=[row_spec, row_spec, full(wlw_t), full(blw_r), full(wcat),
                  full(wsum_t), full(b0p), full(w1p), full(b1p)],
        out_specs=pl.BlockSpec((8, tb), lambda i: (0, i)),
        scratch_shapes=[pltpu.VMEM((2 * md_pad + groups, 2 * d), jnp.bfloat16),
                        pltpu.VMEM((d, in_dim), f32),
                        pltpu.VMEM((d + b0p.shape[1], 1), f32)],
        compiler_params=pltpu.CompilerParams(
            dimension_semantics=("arbitrary",),
            vmem_limit_bytes=100 << 20,
        ),
    )(x1, x2, wlw_t, blw_r, wcat, wsum_t, b0p, w1p, b1p)
    return out[0:1, :batch].T


# speedup vs baseline: 1.0951x; 1.0152x over previous
import functools

import jax
import jax.numpy as jnp
from jax import lax
from jax.experimental import pallas as pl
from jax.experimental.pallas import tpu as pltpu


def _round_up(x, m):
    return (x + m - 1) // m * m


def _body(x1_ref, x2_ref, wlwt_ref, blw_ref, wcat_ref, wsum_ref, b0_ref,
          w1_ref, b1_ref, out_ref, wbig_ref, wlw_s, colv_s, *, d, groups, tb):
    f32 = jnp.float32
    bf16 = jnp.bfloat16
    md = groups * d

    @pl.when(pl.program_id(0) == 0)
    def _build_weights():
        t_ab = jnp.transpose(wcat_ref[:, :md].astype(bf16))
        t_s = jnp.transpose(wcat_ref[:, md:2 * md].astype(bf16))
        t_sum = jnp.transpose(wsum_ref[:, :groups].astype(bf16))
        wbig_ref[:md, :d] = t_ab
        wbig_ref[:md, d:] = t_s
        wbig_ref[md:2 * md, :d] = t_s
        wbig_ref[md:2 * md, d:] = jnp.zeros((md, d), bf16)
        wbig_ref[2 * md:, :d] = t_sum
        wbig_ref[2 * md:, d:] = t_sum
        wlw_s[...] = jnp.transpose(wlwt_ref[...])
        colv_s[:d] = jnp.transpose(blw_ref[...])
        colv_s[d:] = jnp.transpose(b0_ref[...])

    blw = colv_s[:d]
    wlwt = wlw_s[...]

    def frontend(x_ref):
        y = lax.dot_general(wlwt, x_ref[...], (((1,), (1,)), ((), ())),
                            preferred_element_type=f32) + blw
        ss = jnp.sum(y * y, axis=0, keepdims=True)
        return y * lax.rsqrt(jnp.maximum(ss, 1e-24))

    at = frontend(x1_ref)
    bt = frontend(x2_ref)
    ct = jnp.concatenate([at, bt], axis=0).astype(bf16)

    big = jnp.dot(wbig_ref[...], ct, preferred_element_type=f32)

    p1 = big[:md].reshape(groups, d, tb)
    p2 = big[md:2 * md].reshape(groups, d, tb)
    sumterm = big[2 * md:2 * md + groups]

    s3 = p1 * bt[None, :, :] + p2 * at[None, :, :]
    b0c = colv_s[d:d + groups]
    h = jnp.sum(s3, axis=1) + sumterm + b0c
    h = jnp.maximum(h, 0.0)

    w1c = w1_ref[:groups, :1]
    score = jnp.sum(h * w1c, axis=0, keepdims=True) + b1_ref[0:1, 0:1]
    out_ref[...] = score.reshape(1, 1, tb)


def kernel(x1, x2, wlw_t, blw_r, wcat, wsum_t, b0p, pool, w1p, b1p):
    batch, in_dim = x1.shape
    d = wlw_t.shape[1]
    md_pad = wcat.shape[1] // 3
    groups = md_pad // d
    f32 = jnp.float32

    tb = 256
    b_pad = _round_up(batch, tb)
    pad = b_pad - batch
    x1 = x1.astype(f32)
    x2 = x2.astype(f32)
    if pad:
        x1 = jnp.pad(x1, ((0, pad), (0, 0)))
        x2 = jnp.pad(x2, ((0, pad), (0, 0)))

    body = functools.partial(_body, d=d, groups=groups, tb=tb)
    row_spec = pl.BlockSpec((tb, in_dim), lambda i: (i, 0))
    full = lambda w: pl.BlockSpec(tuple(w.shape), lambda i: (0, 0))

    out = pl.pallas_call(
        body,
        out_shape=jax.ShapeDtypeStruct((b_pad // tb, 1, tb), f32),
        grid=(b_pad // tb,),
        in_specs=[row_spec, row_spec, full(wlw_t), full(blw_r), full(wcat),
                  full(wsum_t), full(b0p), full(w1p), full(b1p)],
        out_specs=pl.BlockSpec((1, 1, tb), lambda i: (i, 0, 0)),
        scratch_shapes=[pltpu.VMEM((2 * md_pad + groups, 2 * d), jnp.bfloat16),
                        pltpu.VMEM((d, in_dim), f32),
                        pltpu.VMEM((d + b0p.shape[1], 1), f32)],
        compiler_params=pltpu.CompilerParams(
            dimension_semantics=("arbitrary",),
            vmem_limit_bytes=100 << 20,
        ),
    )(x1, x2, wlw_t, blw_r, wcat, wsum_t, b0p, w1p, b1p)
    return out.reshape(b_pad, 1)[:batch]
